# Initial kernel scaffold; baseline (speedup 1.0000x reference)
import jax
import jax.numpy as jnp
from jax import lax
from jax.experimental import pallas as pl
from jax.experimental.pallas import tpu as pltpu
from jax.experimental.pallas import tpu_sc as plsc

NUM_EMBEDDINGS = 1000000
FEATURES = 32
BATCH = 16384
SEQ = 20

TOTAL = BATCH * SEQ
LANE = 128
ROWS = TOTAL // LANE
NW = 32
ROWS_PER_W = ROWS // NW
CHUNK_ROWS = 16
NCHUNK = ROWS_PER_W // CHUNK_ROWS


def _body(idx_hbm, table_hbm, out_hbm, idx_v, rows_v, gsem):
    wid = lax.axis_index("s") * 2 + lax.axis_index("c")
    base = wid * ROWS_PER_W

    def chunk(c, _):
        row0 = base + c * CHUNK_ROWS
        pltpu.sync_copy(idx_hbm.at[pl.ds(row0, CHUNK_ROWS)], idx_v)
        pltpu.async_copy(table_hbm.at[idx_v], rows_v, gsem).wait()
        pltpu.sync_copy(rows_v, out_hbm.at[pl.ds(row0, CHUNK_ROWS)])
        return ()

    lax.fori_loop(0, NCHUNK, chunk, ())


def kernel(inputs, embedding):
    idx = inputs.reshape(ROWS, LANE)
    mesh = plsc.VectorSubcoreMesh(core_axis_name="c", subcore_axis_name="s")
    out = pl.kernel(
        _body,
        mesh=mesh,
        out_type=jax.ShapeDtypeStruct((ROWS, LANE, FEATURES), jnp.float32),
        scratch_types=[
            pltpu.VMEM((CHUNK_ROWS, LANE), jnp.int32),
            pltpu.VMEM((CHUNK_ROWS, LANE, FEATURES), jnp.float32),
            pltpu.SemaphoreType.DMA,
        ],
    )(idx, embedding)
    return out.reshape(BATCH, SEQ, FEATURES)


# speedup vs baseline: 1.2720x; 1.2720x over previous
import jax
import jax.numpy as jnp
from jax import lax
from jax.experimental import pallas as pl
from jax.experimental.pallas import tpu as pltpu
from jax.experimental.pallas import tpu_sc as plsc

NUM_EMBEDDINGS = 1000000
FEATURES = 32
BATCH = 16384
SEQ = 20

TOTAL = BATCH * SEQ
LANE = 128
ROWS = TOTAL // LANE
NW = 32
ROWS_PER_W = ROWS // NW
CHUNK_ROWS = 16
NCHUNK = ROWS_PER_W // CHUNK_ROWS


def _body(idx_hbm, table_hbm, out_hbm, idx_v, rows_v, gsem):
    wid = lax.axis_index("s") * 2 + lax.axis_index("c")
    base = wid * ROWS_PER_W

    def chunk(c, _):
        row0 = base + c * CHUNK_ROWS
        pltpu.sync_copy(idx_hbm.at[pl.ds(row0, CHUNK_ROWS)], idx_v)
        copies = [
            pltpu.async_copy(table_hbm.at[idx_v.at[j]], rows_v.at[j], gsem)
            for j in range(CHUNK_ROWS)
        ]
        for cp in copies:
            cp.wait()
        pltpu.sync_copy(rows_v, out_hbm.at[pl.ds(row0, CHUNK_ROWS)])
        return ()

    lax.fori_loop(0, NCHUNK, chunk, ())


def kernel(inputs, embedding):
    idx = inputs.reshape(ROWS, LANE)
    mesh = plsc.VectorSubcoreMesh(core_axis_name="c", subcore_axis_name="s")
    out = pl.kernel(
        _body,
        mesh=mesh,
        out_type=jax.ShapeDtypeStruct((ROWS, LANE, FEATURES), jnp.float32),
        scratch_types=[
            pltpu.VMEM((CHUNK_ROWS, LANE), jnp.int32),
            pltpu.VMEM((CHUNK_ROWS, LANE, FEATURES), jnp.float32),
            pltpu.SemaphoreType.DMA,
        ],
        compiler_params=pltpu.CompilerParams(use_tc_tiling_on_sc=False),
    )(idx, embedding)
    return out.reshape(BATCH, SEQ, FEATURES)


# speedup vs baseline: 1.2783x; 1.0050x over previous
import jax
import jax.numpy as jnp
from jax import lax
from jax.experimental import pallas as pl
from jax.experimental.pallas import tpu as pltpu
from jax.experimental.pallas import tpu_sc as plsc

NUM_EMBEDDINGS = 1000000
FEATURES = 32
BATCH = 16384
SEQ = 20

TOTAL = BATCH * SEQ
LANE = 128
ROWS = TOTAL // LANE
NW = 32
ROWS_PER_W = ROWS // NW
CHUNK_ROWS = 8
NCHUNK = ROWS_PER_W // CHUNK_ROWS
NBUF = 2
NGROUP = NCHUNK // NBUF


def _body(idx_hbm, table_hbm, out_hbm, idx_v, rows_v, gsem, osem, isem):
    wid = lax.axis_index("s") * 2 + lax.axis_index("c")
    base = wid * ROWS_PER_W

    def start_idx_load(c, b):
        row0 = base + c * CHUNK_ROWS
        pltpu.async_copy(idx_hbm.at[pl.ds(row0, CHUNK_ROWS)], idx_v.at[b],
                         isem.at[b])

    def wait_idx_load(c, b):
        row0 = base + c * CHUNK_ROWS
        pltpu.make_async_copy(idx_hbm.at[pl.ds(row0, CHUNK_ROWS)],
                              idx_v.at[b], isem.at[b]).wait()

    def issue_gathers(b):
        for j in range(CHUNK_ROWS):
            pltpu.async_copy(table_hbm.at[idx_v.at[b, j]], rows_v.at[b, j],
                             gsem.at[b])

    def wait_gathers(c, b):
        for j in range(CHUNK_ROWS):
            pltpu.make_async_copy(table_hbm.at[idx_v.at[b, j]],
                                  rows_v.at[b, j], gsem.at[b]).wait()

    def start_writeback(c, b):
        row0 = base + c * CHUNK_ROWS
        pltpu.async_copy(rows_v.at[b], out_hbm.at[pl.ds(row0, CHUNK_ROWS)],
                         osem.at[b])

    def wait_writeback(c, b):
        row0 = base + c * CHUNK_ROWS
        pltpu.make_async_copy(rows_v.at[b],
                              out_hbm.at[pl.ds(row0, CHUNK_ROWS)],
                              osem.at[b]).wait()

    for b in range(NBUF):
        start_idx_load(b, b)
        wait_idx_load(b, b)
        issue_gathers(b)

    def group(g, _):
        for b in range(NBUF):
            c = g * NBUF + b
            wait_gathers(c, b)
            start_idx_load(c + NBUF, b)
            start_writeback(c, b)
            wait_writeback(c, b)
            wait_idx_load(c + NBUF, b)
            issue_gathers(b)
        return ()

    lax.fori_loop(0, NGROUP - 1, group, ())

    for b in range(NBUF):
        c = (NGROUP - 1) * NBUF + b
        wait_gathers(c, b)
        start_writeback(c, b)
        wait_writeback(c, b)


def kernel(inputs, embedding):
    idx = inputs.reshape(ROWS, LANE)
    mesh = plsc.VectorSubcoreMesh(core_axis_name="c", subcore_axis_name="s")
    out = pl.kernel(
        _body,
        mesh=mesh,
        out_type=jax.ShapeDtypeStruct((ROWS, LANE, FEATURES), jnp.float32),
        scratch_types=[
            pltpu.VMEM((NBUF, CHUNK_ROWS, LANE), jnp.int32),
            pltpu.VMEM((NBUF, CHUNK_ROWS, LANE, FEATURES), jnp.float32),
            pltpu.SemaphoreType.DMA((NBUF,)),
            pltpu.SemaphoreType.DMA((NBUF,)),
            pltpu.SemaphoreType.DMA((NBUF,)),
        ],
        compiler_params=pltpu.CompilerParams(use_tc_tiling_on_sc=False),
    )(idx, embedding)
    return out.reshape(BATCH, SEQ, FEATURES)
